# Initial kernel scaffold; baseline (speedup 1.0000x reference)
import jax
import jax.numpy as jnp
from jax.experimental import pallas as pl


def kernel(in_p, out_p, f, lift_w1, lift_b1, lift_w2, lift_b2, sw_r, sw_i, skip_w, skip_b, gno_w1, gno_b1, gno_w2, gno_b2, gno_w3, gno_b3, proj_w1, proj_b1, proj_w2, proj_b2):
    raise NotImplementedError("write your pallas kernel here")


# speedup vs baseline: 2.9794x; 2.9794x over previous
import jax
import jax.numpy as jnp
from jax.experimental import pallas as pl

_MODES = 8
_HID = 32
_GRID = 32
_RADIUS = 0.033
_NL = 4
_K = 128
_QB = 32
_LB = 2048
_PB = 2048


def _mlp2_body(x_ref, w1_ref, b1_ref, w2_ref, b2_ref, o_ref):
    h = jnp.dot(x_ref[...], w1_ref[...], preferred_element_type=jnp.float32) + b1_ref[...]
    h = jax.nn.gelu(h)
    o_ref[...] = jnp.dot(h, w2_ref[...], preferred_element_type=jnp.float32) + b2_ref[...]


def _mlp2(x, w1t, b1, w2t, b2, block):
    n, cin = x.shape
    h1 = w1t.shape[1]
    cout = w2t.shape[1]
    return pl.pallas_call(
        _mlp2_body,
        grid=(n // block,),
        in_specs=[
            pl.BlockSpec((block, cin), lambda i: (i, 0)),
            pl.BlockSpec((cin, h1), lambda i: (0, 0)),
            pl.BlockSpec((1, h1), lambda i: (0, 0)),
            pl.BlockSpec((h1, cout), lambda i: (0, 0)),
            pl.BlockSpec((1, cout), lambda i: (0, 0)),
        ],
        out_specs=pl.BlockSpec((block, cout), lambda i: (i, 0)),
        out_shape=jax.ShapeDtypeStruct((n, cout), jnp.float32),
    )(x, w1t, b1.reshape(1, -1), w2t, b2.reshape(1, -1))


def _gno_body(kin_ref, lg_ref, mf_ref, w1_ref, b1_ref, w2_ref, b2_ref,
              w3_ref, b3_ref, o_ref):
    h = jax.nn.gelu(jnp.dot(kin_ref[...], w1_ref[...],
                            preferred_element_type=jnp.float32) + b1_ref[...])
    h = jax.nn.gelu(jnp.dot(h, w2_ref[...],
                            preferred_element_type=jnp.float32) + b2_ref[...])
    k = jnp.dot(h, w3_ref[...], preferred_element_type=jnp.float32) + b3_ref[...]
    mf = mf_ref[...]
    msg = mf * k * lg_ref[...]
    rows = _QB * _K
    qid = jax.lax.broadcasted_iota(jnp.int32, (_QB, rows), 0)
    rid = jax.lax.broadcasted_iota(jnp.int32, (_QB, rows), 1) // _K
    seg = (qid == rid).astype(jnp.float32)
    agg = jnp.dot(seg, msg, preferred_element_type=jnp.float32)
    cnt = jnp.dot(seg, mf, preferred_element_type=jnp.float32)
    o_ref[...] = agg / jnp.maximum(cnt, 1.0)


def _gno(kin, lg, mf, w1t, b1, w2t, b2, w3t, b3, n_q):
    rows = _QB * _K
    return pl.pallas_call(
        _gno_body,
        grid=(n_q // _QB,),
        in_specs=[
            pl.BlockSpec((rows, 6), lambda i: (i, 0)),
            pl.BlockSpec((rows, _HID), lambda i: (i, 0)),
            pl.BlockSpec((rows, 1), lambda i: (i, 0)),
            pl.BlockSpec((6, 512), lambda i: (0, 0)),
            pl.BlockSpec((1, 512), lambda i: (0, 0)),
            pl.BlockSpec((512, 256), lambda i: (0, 0)),
            pl.BlockSpec((1, 256), lambda i: (0, 0)),
            pl.BlockSpec((256, _HID), lambda i: (0, 0)),
            pl.BlockSpec((1, _HID), lambda i: (0, 0)),
        ],
        out_specs=pl.BlockSpec((_QB, _HID), lambda i: (i, 0)),
        out_shape=jax.ShapeDtypeStruct((n_q, _HID), jnp.float32),
    )(kin, lg, mf, w1t, b1.reshape(1, -1), w2t, b2.reshape(1, -1),
      w3t, b3.reshape(1, -1))


def kernel(in_p, out_p, f, lift_w1, lift_b1, lift_w2, lift_b2, sw_r, sw_i,
           skip_w, skip_b, gno_w1, gno_b1, gno_w2, gno_b2, gno_w3, gno_b3,
           proj_w1, proj_b1, proj_w2, proj_b2):
    g = _GRID
    m = _MODES

    xin = jnp.concatenate([f, in_p], axis=-1).reshape(-1, 6)
    lifted = _mlp2(xin, lift_w1.T, lift_b1, lift_w2.T, lift_b2, _LB)
    x = lifted.reshape(g, g, g, _HID).transpose(3, 0, 1, 2)

    corners = [(slice(0, m), slice(0, m)), (slice(0, m), slice(-m, None)),
               (slice(-m, None), slice(0, m)), (slice(-m, None), slice(-m, None))]
    for l in range(_NL):
        x_ft = jnp.fft.rfftn(x, axes=(1, 2, 3), norm='forward')
        out_ft = jnp.zeros_like(x_ft)
        for ci, (s1, s2) in enumerate(corners):
            w = sw_r[l, ci] + 1j * sw_i[l, ci]
            blk = x_ft[:, s1, s2, :m]
            out_ft = out_ft.at[:, s1, s2, :m].set(
                jnp.einsum('ixyz,ioxyz->oxyz', blk, w))
        x_sc = jnp.fft.irfftn(out_ft, s=(g, g, g), axes=(1, 2, 3), norm='forward')
        x_skip = jnp.einsum('oi,ixyz->oxyz', skip_w[l], x) + \
            skip_b[l][:, None, None, None]
        x = x_sc + x_skip
        if l < _NL - 1:
            x = jax.nn.gelu(x)

    latent = jnp.transpose(x, (1, 2, 3, 0)).reshape(-1, _HID)

    y = in_p.reshape(-1, 3)
    n_q = out_p.shape[0]
    y2 = jnp.sum(y * y, axis=1)

    def _sel(xc):
        d2 = jnp.sum(xc * xc, axis=1)[:, None] + y2[None, :] - 2.0 * (xc @ y.T)
        _, idx = jax.lax.top_k(-d2, _K)
        return idx

    idx = jax.lax.map(_sel, out_p.reshape(8, -1, 3)).reshape(-1, _K)

    yg = y[idx]
    diff = out_p[:, None, :] - yg
    d2e = jnp.sum(diff * diff, axis=-1)
    mf = (d2e < _RADIUS * _RADIUS).astype(jnp.float32).reshape(-1, 1)
    kin = jnp.concatenate(
        [yg, jnp.broadcast_to(out_p[:, None, :], yg.shape)],
        axis=-1).reshape(-1, 6)
    lg = latent[idx].reshape(-1, _HID)

    agg = _gno(kin, lg, mf, gno_w1.T, gno_b1, gno_w2.T, gno_b2,
               gno_w3.T, gno_b3, n_q)

    return _mlp2(agg, proj_w1.T, proj_b1, proj_w2.T, proj_b2, _PB)
